# baseline (speedup 1.0000x reference)
import jax
import jax.numpy as jnp
from jax import lax
from jax.experimental import pallas as pl
from jax.experimental.pallas import tpu as pltpu
from jax.experimental.pallas import tpu_sc as plsc

_B = 1024
_S = 26
_D = 1000
_K0 = 504
_NC = 2
_NS = 16
_NW = _NC * _NS
_IT = _B // 128
_NU = _S * _IT
_UPW = -(-_NU // _NW)
_L = 16
_GR = 128 // _L
_HALF = ((0, _K0), (_K0, _D - _K0))


def _sc_body(xt_hbm, out_hbm, idx_v, buf0, buf1, isem, sem0, sem1):
    bufs = (buf0, buf1)
    sems = (sem0, sem1)
    wid = lax.axis_index("s") * _NC + lax.axis_index("c")

    def unit(u):
        uid = wid + _NW * u
        return uid, uid // _IT, lax.rem(uid, _IT)

    for u in range(_UPW):
        uid, j, it = unit(u)

        @pl.when(uid < _NU)
        def _prefetch():
            pltpu.async_copy(
                xt_hbm.at[j, pl.ds(it * 128, 128)],
                idx_v.at[pl.ds(u * 128, 128)],
                isem,
            )

    lanes = lax.iota(jnp.int32, _L)
    zeros16 = jnp.zeros((_L,), jnp.float32)
    ones16 = jnp.ones((_L,), jnp.float32)

    for h in range(2):
        def zrow(r, c, h=h):
            for g in range(_GR):
                bufs[h][r, pl.ds(g * _L, _L)] = zeros16
            return c

        lax.fori_loop(0, _HALF[h][1], zrow, 0)

    for u in range(_UPW):
        uid, j, it = unit(u)

        @pl.when(uid < _NU)
        def _drain_idx():
            pltpu.make_async_copy(
                xt_hbm.at[j, pl.ds(it * 128, 128)],
                idx_v.at[pl.ds(u * 128, 128)],
                isem,
            ).wait()

    def scatter(h, u, val16):
        k0, nk = _HALF[h]
        for g in range(_GR):
            xv = idx_v[pl.ds(u * 128 + g * _L, _L)]
            row = xv - k0
            mask = (xv >= k0) & (xv < k0 + nk)
            plsc.store_scatter(bufs[h], [row, g * _L + lanes], val16, mask=mask)

    def dst(h, j, it):
        k0, nk = _HALF[h]
        return out_hbm.at[j, pl.ds(k0, nk), pl.ds(it * 128, 128)]

    for u in range(_UPW):
        uid, j, it = unit(u)

        @pl.when(uid < _NU)
        def _do_unit():
            for h in range(2):
                if u >= 1:
                    puid, pj, pit = unit(u - 1)
                    pltpu.make_async_copy(
                        bufs[h], dst(h, pj, pit), sems[h]
                    ).wait()
                    scatter(h, u - 1, zeros16)
                scatter(h, u, ones16)
                pltpu.async_copy(bufs[h], dst(h, j, it), sems[h])

    for u in (_UPW - 2, _UPW - 1):
        uid, j, it = unit(u)
        nxt = unit(u + 1)[0] if u + 1 < _UPW else _NU

        @pl.when((uid < _NU) & (nxt >= _NU))
        def _drain(j=j, it=it):
            for h in range(2):
                pltpu.make_async_copy(bufs[h], dst(h, j, it), sems[h]).wait()


def _one_hot(xt):
    mesh = plsc.VectorSubcoreMesh(core_axis_name="c", subcore_axis_name="s")
    f = pl.kernel(
        _sc_body,
        out_type=jax.ShapeDtypeStruct((_S, _D, _B), jnp.float32),
        mesh=mesh,
        scratch_types=[
            pltpu.VMEM((_UPW * 128,), jnp.int32),
            pltpu.VMEM((_HALF[0][1], 128), jnp.float32),
            pltpu.VMEM((_HALF[1][1], 128), jnp.float32),
            pltpu.SemaphoreType.DMA,
            pltpu.SemaphoreType.DMA,
            pltpu.SemaphoreType.DMA,
        ],
        compiler_params=pltpu.CompilerParams(
            needs_layout_passes=False,
            skip_device_barrier=True,
            disable_semaphore_checks=True,
        ),
    )
    return f(xt)


def kernel(x, ones):
    xt = jnp.transpose(x.astype(jnp.int32))
    out3 = _one_hot(xt)
    return jnp.transpose(out3, (2, 0, 1))


# speedup vs baseline: 1.0337x; 1.0337x over previous
import jax
import jax.numpy as jnp
from jax import lax
from jax.experimental import pallas as pl
from jax.experimental.pallas import tpu as pltpu
from jax.experimental.pallas import tpu_sc as plsc

_B = 1024
_S = 26
_D = 1000
_K0 = 504
_NC = 2
_NS = 16
_NW = _NC * _NS
_IT = _B // 128
_NU = _S * _IT
_UPW = -(-_NU // _NW)
_L = 16
_GR = 128 // _L
_HALF = ((0, _K0), (_K0, _D - _K0))


def _sc_body(xt_hbm, out_hbm, idx_v, buf0, buf1, isem, sem0, sem1):
    bufs = (buf0, buf1)
    sems = (sem0, sem1)
    wid = lax.axis_index("s") * _NC + lax.axis_index("c")

    def unit(u):
        uid = wid + _NW * u
        return uid, uid // _IT, lax.rem(uid, _IT)

    for u in range(_UPW):
        uid, j, it = unit(u)

        @pl.when(uid < _NU)
        def _prefetch():
            pltpu.async_copy(
                xt_hbm.at[j, pl.ds(it * 128, 128)],
                idx_v.at[pl.ds(u * 128, 128)],
                isem,
            )

    lanes = lax.iota(jnp.int32, _L)
    zeros16 = jnp.zeros((_L,), jnp.float32)
    ones16 = jnp.ones((_L,), jnp.float32)

    def zero_half(h):
        def zrow(r, c):
            for g in range(_GR):
                bufs[h][r, pl.ds(g * _L, _L)] = zeros16
            return c

        lax.fori_loop(0, _HALF[h][1], zrow, 0)

    def drain_idx():
        for u in range(_UPW):
            uid, j, it = unit(u)

            @pl.when(uid < _NU)
            def _drain_idx():
                pltpu.make_async_copy(
                    xt_hbm.at[j, pl.ds(it * 128, 128)],
                    idx_v.at[pl.ds(u * 128, 128)],
                    isem,
                ).wait()

    def scatter(h, u, val16):
        k0, nk = _HALF[h]
        for g in range(_GR):
            xv = idx_v[pl.ds(u * 128 + g * _L, _L)]
            row = xv - k0
            mask = (xv >= k0) & (xv < k0 + nk)
            plsc.store_scatter(bufs[h], [row, g * _L + lanes], val16, mask=mask)

    def dst(h, j, it):
        k0, nk = _HALF[h]
        return out_hbm.at[j, pl.ds(k0, nk), pl.ds(it * 128, 128)]

    zero_half(0)
    drain_idx()
    uid0, j0, it0 = unit(0)
    scatter(0, 0, ones16)
    pltpu.async_copy(bufs[0], dst(0, j0, it0), sems[0])
    zero_half(1)
    scatter(1, 0, ones16)
    pltpu.async_copy(bufs[1], dst(1, j0, it0), sems[1])

    for u in range(1, _UPW):
        uid, j, it = unit(u)

        @pl.when(uid < _NU)
        def _do_unit():
            for h in range(2):
                puid, pj, pit = unit(u - 1)
                pltpu.make_async_copy(
                    bufs[h], dst(h, pj, pit), sems[h]
                ).wait()
                scatter(h, u - 1, zeros16)
                scatter(h, u, ones16)
                pltpu.async_copy(bufs[h], dst(h, j, it), sems[h])

    for u in (_UPW - 2, _UPW - 1):
        uid, j, it = unit(u)
        nxt = unit(u + 1)[0] if u + 1 < _UPW else _NU

        @pl.when((uid < _NU) & (nxt >= _NU))
        def _drain(j=j, it=it):
            for h in range(2):
                pltpu.make_async_copy(bufs[h], dst(h, j, it), sems[h]).wait()


def _one_hot(xt):
    mesh = plsc.VectorSubcoreMesh(core_axis_name="c", subcore_axis_name="s")
    f = pl.kernel(
        _sc_body,
        out_type=jax.ShapeDtypeStruct((_S, _D, _B), jnp.float32),
        mesh=mesh,
        scratch_types=[
            pltpu.VMEM((_UPW * 128,), jnp.int32),
            pltpu.VMEM((_HALF[0][1], 128), jnp.float32),
            pltpu.VMEM((_HALF[1][1], 128), jnp.float32),
            pltpu.SemaphoreType.DMA,
            pltpu.SemaphoreType.DMA,
            pltpu.SemaphoreType.DMA,
        ],
        compiler_params=pltpu.CompilerParams(
            needs_layout_passes=False,
            skip_device_barrier=True,
            disable_semaphore_checks=True,
        ),
    )
    return f(xt)


def kernel(x, ones):
    xt = jnp.transpose(x.astype(jnp.int32))
    out3 = _one_hot(xt)
    return jnp.transpose(out3, (2, 0, 1))
